# baseline (speedup 1.0000x reference)
import functools

import jax
import jax.numpy as jnp
from jax import lax
from jax.experimental import pallas as pl
from jax.experimental.pallas import tpu as pltpu
from jax.experimental.pallas import tpu_sc as plsc

N = 10000
D = 128
_c = jnp.int32
E = 320000
NC = 2
NS = 16
LANES = 16
NW = NC * NS
NPAD = 10240
EB = 128
BLKS = 80
EPT = EB * BLKS
EPAD = EPT * NW
RPT = NPAD // NS
BR = 2048
GRID = NPAD // BR

@functools.lru_cache(maxsize=None)
def _mesh():
    return plsc.VectorSubcoreMesh(
        core_axis_name="c", subcore_axis_name="s",
        num_cores=NC, num_subcores=NS)


def _deg_body(dst_hbm, out_hbm, didx, deg):
    cid = lax.axis_index("c")
    sid = lax.axis_index("s")
    wid = cid * _c(NS) + sid
    pltpu.sync_copy(dst_hbm.at[wid], didx)
    zeros16 = jnp.zeros((LANES,), jnp.float32)

    def zstep(i, carry):
        deg[pl.ds(i * _c(LANES), LANES)] = zeros16
        return carry

    lax.fori_loop(_c(0), _c(NPAD // LANES), zstep, _c(0))
    ones16 = jnp.ones((LANES,), jnp.float32)

    def blk(j, carry):
        for k in range(EB // LANES):
            idx = didx[j, pl.ds(k * LANES, LANES)]
            plsc.addupdate_scatter(deg, [idx], ones16)
        return carry

    lax.fori_loop(_c(0), _c(BLKS), blk, _c(0))
    pltpu.sync_copy(deg, out_hbm.at[wid])


@functools.lru_cache(maxsize=None)
def _deg_call():
    return pl.kernel(
        _deg_body,
        name="sc_deg",
        out_type=jax.ShapeDtypeStruct((NW, NPAD), jnp.float32),
        mesh=_mesh(),
        compiler_params=pltpu.CompilerParams(needs_layout_passes=False, use_tc_tiling_on_sc=False),
        scratch_types=[
            pltpu.VMEM((BLKS, EB), jnp.int32),
            pltpu.VMEM((NPAD,), jnp.float32),
        ],
    )


NBUF = 8


def _stage_zero_loop(C, sid, agg, gsh, g_hbm, r0):
    rbase = sid * _c(RPT)
    pltpu.sync_copy(g_hbm.at[pl.ds(rbase, RPT)], gsh.at[pl.ds(rbase, RPT)])
    zeros16 = jnp.zeros((LANES,), jnp.float32)

    def zrow(r, carry):
        for c in range(C // LANES):
            r0[r, pl.ds(c * LANES, LANES)] = zeros16
        return carry

    lax.fori_loop(_c(0), _c(EB), zrow, _c(0))
    for t in range(RPT // EB):
        pltpu.sync_copy(r0, agg.at[pl.ds(rbase + _c(t * EB), EB)])


BLK2 = BLKS


def _edge_loop(sidx, didx, rows, gsem, ssem, gsh, agg):
    def sl(ref, j):
        return ref.at[j]

    for j in range(NBUF - 1):
        pltpu.async_copy(gsh.at[sl(sidx, _c(j))], rows[j], gsem[j])

    def step(i, carry):
        base = i * _c(NBUF)
        for b in range(NBUF):
            j = base + _c(b)
            jj = j + _c(NBUF - 1)
            bb = (b + NBUF - 1) % NBUF

            @pl.when(jj < _c(BLK2))
            def _():
                @pl.when(jj >= _c(NBUF))
                def _():
                    pltpu.make_async_copy(
                        rows[bb], agg.at[sl(didx, _c(0))], ssem[bb]).wait()

                pltpu.async_copy(gsh.at[sl(sidx, jj)], rows[bb], gsem[bb])

            pltpu.make_async_copy(gsh.at[sl(sidx, j)], rows[b],
                                  gsem[b]).wait()
            pltpu.async_copy(rows[b], agg.at[sl(didx, j)], ssem[b], add=True)
        return carry

    lax.fori_loop(_c(0), _c(BLK2 // NBUF), step, _c(0))
    for b in range(NBUF):
        pltpu.make_async_copy(rows[b], agg.at[sl(didx, _c(0))], ssem[b]).wait()


def _prop_body(C, g_hbm, src_hbm, dst_hbm, out_hbm, sidx, didx, *rest):
    rows = rest[:NBUF]
    agg, gsh = rest[NBUF], rest[NBUF + 1]
    gsem = rest[NBUF + 2:2 * NBUF + 2]
    ssem = rest[2 * NBUF + 2:]
    cid = lax.axis_index("c")
    sid = lax.axis_index("s")
    wid = cid * _c(NS) + sid
    pltpu.sync_copy(src_hbm.at[wid], sidx)
    pltpu.sync_copy(dst_hbm.at[wid], didx)
    _stage_zero_loop(C, sid, agg, gsh, g_hbm, rows[0])
    plsc.subcore_barrier()
    _edge_loop(sidx, didx, rows, gsem, ssem, gsh, agg)
    plsc.subcore_barrier()
    rbase = sid * _c(RPT)
    pltpu.sync_copy(agg.at[pl.ds(rbase, RPT)],
                    out_hbm.at[cid, pl.ds(rbase, RPT)])


@functools.lru_cache(maxsize=None)
def _make_prop(C):
    return pl.kernel(
        functools.partial(_prop_body, C),
        name=f"sc_prop{C}",
        out_type=jax.ShapeDtypeStruct((NC, NPAD, C), jnp.float32),
        mesh=_mesh(),
        compiler_params=pltpu.CompilerParams(use_tc_tiling_on_sc=False),
        scratch_types=(
            [pltpu.VMEM((BLKS, EB), jnp.int32)] * 2
            + [pltpu.VMEM((EB, C), jnp.float32)] * NBUF
            + [pltpu.VMEM_SHARED((NPAD, C), jnp.float32)] * 2
            + [pltpu.SemaphoreType.DMA] * (2 * NBUF)
        ),
    )


def _prop64_body(ga_hbm, gb_hbm, src_hbm, dst_hbm, outa_hbm, outb_hbm,
                 sidx, didx, *rest):
    C = 32
    rows = rest[:NBUF]
    agg, gsh = rest[NBUF], rest[NBUF + 1]
    gsem = rest[NBUF + 2:2 * NBUF + 2]
    ssem = rest[2 * NBUF + 2:]
    cid = lax.axis_index("c")
    sid = lax.axis_index("s")
    wid = cid * _c(NS) + sid
    pltpu.sync_copy(src_hbm.at[wid], sidx)
    pltpu.sync_copy(dst_hbm.at[wid], didx)
    rbase = sid * _c(RPT)
    for g_hbm, out_hbm in ((ga_hbm, outa_hbm), (gb_hbm, outb_hbm)):
        _stage_zero_loop(C, sid, agg, gsh, g_hbm, rows[0])
        plsc.subcore_barrier()
        _edge_loop(sidx, didx, rows, gsem, ssem, gsh, agg)
        plsc.subcore_barrier()
        pltpu.sync_copy(agg.at[pl.ds(rbase, RPT)],
                        out_hbm.at[cid, pl.ds(rbase, RPT)])


@functools.lru_cache(maxsize=None)
def _prop64_call():
    C = 32
    return pl.kernel(
        _prop64_body,
        name="sc_prop64",
        out_type=(jax.ShapeDtypeStruct((NC, NPAD, C), jnp.float32),
                  jax.ShapeDtypeStruct((NC, NPAD, C), jnp.float32)),
        mesh=_mesh(),
        compiler_params=pltpu.CompilerParams(use_tc_tiling_on_sc=False),
        scratch_types=(
            [pltpu.VMEM((BLKS, EB), jnp.int32)] * 2
            + [pltpu.VMEM((EB, C), jnp.float32)] * NBUF
            + [pltpu.VMEM_SHARED((NPAD, C), jnp.float32)] * 2
            + [pltpu.SemaphoreType.DMA] * (2 * NBUF)
        ),
    )


def _prop1_body(g_hbm, src_hbm, dst_hbm, out_hbm, gv, sidx, didx, acc):
    cid = lax.axis_index("c")
    sid = lax.axis_index("s")
    wid = cid * _c(NS) + sid
    pltpu.sync_copy(g_hbm, gv)
    pltpu.sync_copy(src_hbm.at[wid], sidx)
    pltpu.sync_copy(dst_hbm.at[wid], didx)
    zeros16 = jnp.zeros((LANES,), jnp.float32)

    def zstep(i, carry):
        acc[pl.ds(i * _c(LANES), LANES)] = zeros16
        return carry

    lax.fori_loop(_c(0), _c(NPAD // LANES), zstep, _c(0))

    def blk(j, carry):
        for k in range(EB // LANES):
            si = sidx[j, pl.ds(k * LANES, LANES)]
            di = didx[j, pl.ds(k * LANES, LANES)]
            vals = plsc.load_gather(gv, [si])
            plsc.addupdate_scatter(acc, [di], vals)
        return carry

    lax.fori_loop(_c(0), _c(BLKS), blk, _c(0))
    pltpu.sync_copy(acc, out_hbm.at[wid])


@functools.lru_cache(maxsize=None)
def _prop1_call():
    return pl.kernel(
        _prop1_body,
        name="sc_prop1",
        out_type=jax.ShapeDtypeStruct((NW, NPAD), jnp.float32),
        mesh=_mesh(),
        compiler_params=pltpu.CompilerParams(needs_layout_passes=False, use_tc_tiling_on_sc=False),
        scratch_types=[
            pltpu.VMEM((NPAD,), jnp.float32),
            pltpu.VMEM((BLKS, EB), jnp.int32),
            pltpu.VMEM((BLKS, EB), jnp.int32),
            pltpu.VMEM((NPAD,), jnp.float32),
        ],
    )


def _stage1_tc(deg_ref, x_ref, w_ref, dinv_ref, g_ref):
    deg = jnp.sum(deg_ref[...], axis=0) + 1.0
    dinv = lax.rsqrt(deg)
    dinv_ref[...] = dinv[:, None]
    h = jnp.dot(x_ref[...], w_ref[...], preferred_element_type=jnp.float32)
    g_ref[...] = h * dinv[:, None]


def _stage1(deg_parts, xp, W1):
    return pl.pallas_call(
        _stage1_tc,
        name="tc_stage1",
        grid=(GRID,),
        in_specs=[
            pl.BlockSpec((NW, BR), lambda i: (_c(0), i)),
            pl.BlockSpec((BR, D), lambda i: (i, _c(0))),
            pl.BlockSpec((D, 16), lambda i: (_c(0), _c(0))),
        ],
        out_specs=[
            pl.BlockSpec((BR, 1), lambda i: (i, _c(0))),
            pl.BlockSpec((BR, 16), lambda i: (i, _c(0))),
        ],
        out_shape=[
            jax.ShapeDtypeStruct((NPAD, 1), jnp.float32),
            jax.ShapeDtypeStruct((NPAD, 16), jnp.float32),
        ],
    )(deg_parts, xp, W1)


def _mid_tc(s_ref, g_ref, dinv_ref, b_ref, w_ref, out_ref):
    s = s_ref[...]
    dinv = dinv_ref[...]
    z = (s[0] + s[1] + g_ref[...]) * dinv + b_ref[...]
    z = jnp.maximum(z, 0.0)
    out_ref[...] = jnp.dot(z, w_ref[...],
                           preferred_element_type=jnp.float32) * dinv


def _mid(s_parts, g_prev, dinv, b_prev, W_next, Cp, Cn):
    return pl.pallas_call(
        _mid_tc,
        name=f"tc_mid{Cp}_{Cn}",
        grid=(GRID,),
        in_specs=[
            pl.BlockSpec((NC, BR, Cp), lambda i: (_c(0), i, _c(0))),
            pl.BlockSpec((BR, Cp), lambda i: (i, _c(0))),
            pl.BlockSpec((BR, 1), lambda i: (i, _c(0))),
            pl.BlockSpec((1, Cp), lambda i: (_c(0), _c(0))),
            pl.BlockSpec((Cp, Cn), lambda i: (_c(0), _c(0))),
        ],
        out_specs=pl.BlockSpec((BR, Cn), lambda i: (i, _c(0))),
        out_shape=jax.ShapeDtypeStruct((NPAD, Cn), jnp.float32),
    )(s_parts, g_prev, dinv, b_prev.reshape(1, Cp), W_next)


def _mid_split_tc(sa_ref, sb_ref, g_ref, dinv_ref, b_ref, w_ref, out_ref):
    sa = sa_ref[...]
    sb = sb_ref[...]
    s = jnp.concatenate([sa[0] + sa[1], sb[0] + sb[1]], axis=1)
    dinv = dinv_ref[...]
    z = (s + g_ref[...]) * dinv + b_ref[...]
    z = jnp.maximum(z, 0.0)
    out_ref[...] = jnp.dot(z, w_ref[...],
                           preferred_element_type=jnp.float32) * dinv


def _mid_split(sa, sb, g_prev, dinv, b_prev, W_next, Cp, Cn):
    ch = Cp // 2
    return pl.pallas_call(
        _mid_split_tc,
        name=f"tc_mids{Cp}_{Cn}",
        grid=(GRID,),
        in_specs=[
            pl.BlockSpec((NC, BR, ch), lambda i: (_c(0), i, _c(0))),
            pl.BlockSpec((NC, BR, ch), lambda i: (_c(0), i, _c(0))),
            pl.BlockSpec((BR, Cp), lambda i: (i, _c(0))),
            pl.BlockSpec((BR, 1), lambda i: (i, _c(0))),
            pl.BlockSpec((1, Cp), lambda i: (_c(0), _c(0))),
            pl.BlockSpec((Cp, Cn), lambda i: (_c(0), _c(0))),
        ],
        out_specs=pl.BlockSpec((BR, Cn), lambda i: (i, _c(0))),
        out_shape=jax.ShapeDtypeStruct((NPAD, Cn), jnp.float32),
    )(sa, sb, g_prev, dinv, b_prev.reshape(1, Cp), W_next)


def _final_tc(s_ref, g_ref, dinv_ref, b_ref, out_ref, acc_ref):
    i = pl.program_id(0)
    s = jnp.sum(s_ref[...], axis=0)[:, None]
    z = (s + g_ref[...]) * dinv_ref[...] + b_ref[...]
    z = jnp.maximum(z, 0.0)
    rows = lax.broadcasted_iota(jnp.int32, (BR, 1), 0) + i * _c(BR)
    z = jnp.where(rows < N, z, -jnp.inf)
    m = jnp.max(z)

    @pl.when(i == 0)
    def _():
        acc_ref[0, 0] = m

    @pl.when(i > 0)
    def _():
        acc_ref[0, 0] = jnp.maximum(acc_ref[0, 0], m)

    @pl.when(i == GRID - 1)
    def _():
        p = acc_ref[0, 0]
        v = p - (jnp.log(jnp.exp(p - p)) + p)
        out_ref[...] = jnp.broadcast_to(v, (1, 1))


def _final(s_parts, g4, dinv, b4):
    return pl.pallas_call(
        _final_tc,
        name="tc_final",
        grid=(GRID,),
        in_specs=[
            pl.BlockSpec((NW, BR), lambda i: (_c(0), i)),
            pl.BlockSpec((BR, 1), lambda i: (i, _c(0))),
            pl.BlockSpec((BR, 1), lambda i: (i, _c(0))),
            pl.BlockSpec((1, 1), lambda i: (_c(0), _c(0))),
        ],
        out_specs=pl.BlockSpec((1, 1), lambda i: (_c(0), _c(0))),
        out_shape=jax.ShapeDtypeStruct((1, 1), jnp.float32),
        scratch_shapes=[pltpu.SMEM((1, 1), jnp.float32)],
    )(s_parts, g4, dinv, b4.reshape(1, 1))


def kernel(x, edge_index, W1, b1, W2, b2, W3, b3, W4, b4):
    src = edge_index[0].astype(jnp.int32)
    dst = edge_index[1].astype(jnp.int32)
    pad = EPAD - E
    src3 = jnp.concatenate(
        [src, jnp.zeros((pad,), jnp.int32)]).reshape(NW, BLKS, EB)
    dst3 = jnp.concatenate(
        [dst, jnp.full((pad,), N, jnp.int32)]).reshape(NW, BLKS, EB)
    xp = jnp.concatenate(
        [x.astype(jnp.float32), jnp.zeros((NPAD - N, D), jnp.float32)], axis=0)

    deg_parts = _deg_call()(dst3)
    dinv, g1 = _stage1(deg_parts, xp, W1)
    s1 = _make_prop(16)(g1, src3, dst3)
    g2 = _mid(s1, g1, dinv, b1, W2, 16, 32)
    s2 = _make_prop(32)(g2, src3, dst3)
    g3 = _mid(s2, g2, dinv, b2, W3, 32, 64)
    s3a, s3b = _prop64_call()(g3[:, :32], g3[:, 32:], src3, dst3)
    g4 = _mid_split(s3a, s3b, g3, dinv, b3, W4, 64, 1)
    s4 = _prop1_call()(g4.reshape(NPAD), src3, dst3)
    return _final(s4, g4, dinv, b4)


# speedup vs baseline: 1.0078x; 1.0078x over previous
import functools

import jax
import jax.numpy as jnp
from jax import lax
from jax.experimental import pallas as pl
from jax.experimental.pallas import tpu as pltpu
from jax.experimental.pallas import tpu_sc as plsc

N = 10000
D = 128
_c = jnp.int32
E = 320000
NC = 2
NS = 16
LANES = 16
NW = NC * NS
NPAD = 10240
EB = 128
BLKS = 80
EPT = EB * BLKS
EPAD = EPT * NW
RPT = NPAD // NS
BR = 2048
GRID = NPAD // BR

@functools.lru_cache(maxsize=None)
def _mesh():
    return plsc.VectorSubcoreMesh(
        core_axis_name="c", subcore_axis_name="s",
        num_cores=NC, num_subcores=NS)


def _deg_body(dst_hbm, out_hbm, didx, deg):
    cid = lax.axis_index("c")
    sid = lax.axis_index("s")
    wid = cid * _c(NS) + sid
    pltpu.sync_copy(dst_hbm.at[wid], didx)
    zeros16 = jnp.zeros((LANES,), jnp.float32)

    def zstep(i, carry):
        deg[pl.ds(i * _c(LANES), LANES)] = zeros16
        return carry

    lax.fori_loop(_c(0), _c(NPAD // LANES), zstep, _c(0))
    ones16 = jnp.ones((LANES,), jnp.float32)

    def blk(i, carry):
        for u in range(2):
            j = i * _c(2) + _c(u)
            for k in range(EB // LANES):
                idx = didx[j, pl.ds(k * LANES, LANES)]
                plsc.addupdate_scatter(deg, [idx], ones16)
        return carry

    lax.fori_loop(_c(0), _c(BLKS // 2), blk, _c(0))
    pltpu.sync_copy(deg, out_hbm.at[wid])


@functools.lru_cache(maxsize=None)
def _deg_call():
    return pl.kernel(
        _deg_body,
        name="sc_deg",
        out_type=jax.ShapeDtypeStruct((NW, NPAD), jnp.float32),
        mesh=_mesh(),
        compiler_params=pltpu.CompilerParams(needs_layout_passes=False, use_tc_tiling_on_sc=False),
        scratch_types=[
            pltpu.VMEM((BLKS, EB), jnp.int32),
            pltpu.VMEM((NPAD,), jnp.float32),
        ],
    )


NBUF = 8


def _stage_zero_loop(C, sid, agg, gsh, g_hbm, r0):
    rbase = sid * _c(RPT)
    pltpu.sync_copy(g_hbm.at[pl.ds(rbase, RPT)], gsh.at[pl.ds(rbase, RPT)])
    zeros16 = jnp.zeros((LANES,), jnp.float32)

    def zrow(r, carry):
        for c in range(C // LANES):
            r0[r, pl.ds(c * LANES, LANES)] = zeros16
        return carry

    lax.fori_loop(_c(0), _c(EB), zrow, _c(0))
    for t in range(RPT // EB):
        pltpu.sync_copy(r0, agg.at[pl.ds(rbase + _c(t * EB), EB)])


BLK2 = BLKS


def _edge_loop(sidx, didx, rows, gsem, ssem, gsh, agg):
    def sl(ref, j):
        return ref.at[j]

    for j in range(NBUF - 1):
        pltpu.async_copy(gsh.at[sl(sidx, _c(j))], rows[j], gsem[j])

    def step(i, carry):
        base = i * _c(NBUF)
        for b in range(NBUF):
            j = base + _c(b)
            jj = j + _c(NBUF - 1)
            bb = (b + NBUF - 1) % NBUF

            @pl.when(jj < _c(BLK2))
            def _():
                @pl.when(jj >= _c(NBUF))
                def _():
                    pltpu.make_async_copy(
                        rows[bb], agg.at[sl(didx, _c(0))], ssem[bb]).wait()

                pltpu.async_copy(gsh.at[sl(sidx, jj)], rows[bb], gsem[bb])

            pltpu.make_async_copy(gsh.at[sl(sidx, j)], rows[b],
                                  gsem[b]).wait()
            pltpu.async_copy(rows[b], agg.at[sl(didx, j)], ssem[b], add=True)
        return carry

    lax.fori_loop(_c(0), _c(BLK2 // NBUF), step, _c(0))
    for b in range(NBUF):
        pltpu.make_async_copy(rows[b], agg.at[sl(didx, _c(0))], ssem[b]).wait()


def _prop_body(C, g_hbm, src_hbm, dst_hbm, out_hbm, sidx, didx, *rest):
    rows = rest[:NBUF]
    agg, gsh = rest[NBUF], rest[NBUF + 1]
    gsem = rest[NBUF + 2:2 * NBUF + 2]
    ssem = rest[2 * NBUF + 2:]
    cid = lax.axis_index("c")
    sid = lax.axis_index("s")
    wid = cid * _c(NS) + sid
    pltpu.sync_copy(src_hbm.at[wid], sidx)
    pltpu.sync_copy(dst_hbm.at[wid], didx)
    _stage_zero_loop(C, sid, agg, gsh, g_hbm, rows[0])
    plsc.subcore_barrier()
    _edge_loop(sidx, didx, rows, gsem, ssem, gsh, agg)
    plsc.subcore_barrier()
    rbase = sid * _c(RPT)
    pltpu.sync_copy(agg.at[pl.ds(rbase, RPT)],
                    out_hbm.at[cid, pl.ds(rbase, RPT)])


@functools.lru_cache(maxsize=None)
def _make_prop(C):
    return pl.kernel(
        functools.partial(_prop_body, C),
        name=f"sc_prop{C}",
        out_type=jax.ShapeDtypeStruct((NC, NPAD, C), jnp.float32),
        mesh=_mesh(),
        compiler_params=pltpu.CompilerParams(use_tc_tiling_on_sc=False),
        scratch_types=(
            [pltpu.VMEM((BLKS, EB), jnp.int32)] * 2
            + [pltpu.VMEM((EB, C), jnp.float32)] * NBUF
            + [pltpu.VMEM_SHARED((NPAD, C), jnp.float32)] * 2
            + [pltpu.SemaphoreType.DMA] * (2 * NBUF)
        ),
    )


def _prop64_body(ga_hbm, gb_hbm, src_hbm, dst_hbm, outa_hbm, outb_hbm,
                 sidx, didx, *rest):
    C = 32
    rows = rest[:NBUF]
    agg, gsh = rest[NBUF], rest[NBUF + 1]
    gsem = rest[NBUF + 2:2 * NBUF + 2]
    ssem = rest[2 * NBUF + 2:]
    cid = lax.axis_index("c")
    sid = lax.axis_index("s")
    wid = cid * _c(NS) + sid
    pltpu.sync_copy(src_hbm.at[wid], sidx)
    pltpu.sync_copy(dst_hbm.at[wid], didx)
    rbase = sid * _c(RPT)
    for g_hbm, out_hbm in ((ga_hbm, outa_hbm), (gb_hbm, outb_hbm)):
        _stage_zero_loop(C, sid, agg, gsh, g_hbm, rows[0])
        plsc.subcore_barrier()
        _edge_loop(sidx, didx, rows, gsem, ssem, gsh, agg)
        plsc.subcore_barrier()
        pltpu.sync_copy(agg.at[pl.ds(rbase, RPT)],
                        out_hbm.at[cid, pl.ds(rbase, RPT)])


@functools.lru_cache(maxsize=None)
def _prop64_call():
    C = 32
    return pl.kernel(
        _prop64_body,
        name="sc_prop64",
        out_type=(jax.ShapeDtypeStruct((NC, NPAD, C), jnp.float32),
                  jax.ShapeDtypeStruct((NC, NPAD, C), jnp.float32)),
        mesh=_mesh(),
        compiler_params=pltpu.CompilerParams(use_tc_tiling_on_sc=False),
        scratch_types=(
            [pltpu.VMEM((BLKS, EB), jnp.int32)] * 2
            + [pltpu.VMEM((EB, C), jnp.float32)] * NBUF
            + [pltpu.VMEM_SHARED((NPAD, C), jnp.float32)] * 2
            + [pltpu.SemaphoreType.DMA] * (2 * NBUF)
        ),
    )


def _prop1_body(g_hbm, src_hbm, dst_hbm, out_hbm, gv, sidx, didx, acc):
    cid = lax.axis_index("c")
    sid = lax.axis_index("s")
    wid = cid * _c(NS) + sid
    pltpu.sync_copy(g_hbm, gv)
    pltpu.sync_copy(src_hbm.at[wid], sidx)
    pltpu.sync_copy(dst_hbm.at[wid], didx)
    zeros16 = jnp.zeros((LANES,), jnp.float32)

    def zstep(i, carry):
        acc[pl.ds(i * _c(LANES), LANES)] = zeros16
        return carry

    lax.fori_loop(_c(0), _c(NPAD // LANES), zstep, _c(0))

    def blk(i, carry):
        for u in range(2):
            j = i * _c(2) + _c(u)
            for k in range(EB // LANES):
                si = sidx[j, pl.ds(k * LANES, LANES)]
                di = didx[j, pl.ds(k * LANES, LANES)]
                vals = plsc.load_gather(gv, [si])
                plsc.addupdate_scatter(acc, [di], vals)
        return carry

    lax.fori_loop(_c(0), _c(BLKS // 2), blk, _c(0))
    pltpu.sync_copy(acc, out_hbm.at[wid])


@functools.lru_cache(maxsize=None)
def _prop1_call():
    return pl.kernel(
        _prop1_body,
        name="sc_prop1",
        out_type=jax.ShapeDtypeStruct((NW, NPAD), jnp.float32),
        mesh=_mesh(),
        compiler_params=pltpu.CompilerParams(needs_layout_passes=False, use_tc_tiling_on_sc=False),
        scratch_types=[
            pltpu.VMEM((NPAD,), jnp.float32),
            pltpu.VMEM((BLKS, EB), jnp.int32),
            pltpu.VMEM((BLKS, EB), jnp.int32),
            pltpu.VMEM((NPAD,), jnp.float32),
        ],
    )


def _stage1_tc(deg_ref, x_ref, w_ref, dinv_ref, g_ref):
    deg = jnp.sum(deg_ref[...], axis=0) + 1.0
    dinv = lax.rsqrt(deg)
    dinv_ref[...] = dinv[:, None]
    h = jnp.dot(x_ref[...], w_ref[...], preferred_element_type=jnp.float32)
    g_ref[...] = h * dinv[:, None]


def _stage1(deg_parts, xp, W1):
    return pl.pallas_call(
        _stage1_tc,
        name="tc_stage1",
        grid=(GRID,),
        in_specs=[
            pl.BlockSpec((NW, BR), lambda i: (_c(0), i)),
            pl.BlockSpec((BR, D), lambda i: (i, _c(0))),
            pl.BlockSpec((D, 16), lambda i: (_c(0), _c(0))),
        ],
        out_specs=[
            pl.BlockSpec((BR, 1), lambda i: (i, _c(0))),
            pl.BlockSpec((BR, 16), lambda i: (i, _c(0))),
        ],
        out_shape=[
            jax.ShapeDtypeStruct((NPAD, 1), jnp.float32),
            jax.ShapeDtypeStruct((NPAD, 16), jnp.float32),
        ],
    )(deg_parts, xp, W1)


def _mid_tc(s_ref, g_ref, dinv_ref, b_ref, w_ref, out_ref):
    s = s_ref[...]
    dinv = dinv_ref[...]
    z = (s[0] + s[1] + g_ref[...]) * dinv + b_ref[...]
    z = jnp.maximum(z, 0.0)
    out_ref[...] = jnp.dot(z, w_ref[...],
                           preferred_element_type=jnp.float32) * dinv


def _mid2_tc(s_ref, g_ref, dinv_ref, b_ref, w_ref, outa_ref, outb_ref):
    s = s_ref[...]
    dinv = dinv_ref[...]
    z = (s[0] + s[1] + g_ref[...]) * dinv + b_ref[...]
    z = jnp.maximum(z, 0.0)
    g = jnp.dot(z, w_ref[...], preferred_element_type=jnp.float32) * dinv
    outa_ref[...] = g[:, :32]
    outb_ref[...] = g[:, 32:]


def _mid2(s_parts, g_prev, dinv, b_prev, W_next, Cp, Cn):
    return pl.pallas_call(
        _mid2_tc,
        name=f"tc_mid2_{Cp}_{Cn}",
        grid=(GRID,),
        in_specs=[
            pl.BlockSpec((NC, BR, Cp), lambda i: (_c(0), i, _c(0))),
            pl.BlockSpec((BR, Cp), lambda i: (i, _c(0))),
            pl.BlockSpec((BR, 1), lambda i: (i, _c(0))),
            pl.BlockSpec((1, Cp), lambda i: (_c(0), _c(0))),
            pl.BlockSpec((Cp, Cn), lambda i: (_c(0), _c(0))),
        ],
        out_specs=[
            pl.BlockSpec((BR, Cn // 2), lambda i: (i, _c(0))),
            pl.BlockSpec((BR, Cn // 2), lambda i: (i, _c(0))),
        ],
        out_shape=[
            jax.ShapeDtypeStruct((NPAD, Cn // 2), jnp.float32),
            jax.ShapeDtypeStruct((NPAD, Cn // 2), jnp.float32),
        ],
    )(s_parts, g_prev, dinv, b_prev.reshape(1, Cp), W_next)


def _mid(s_parts, g_prev, dinv, b_prev, W_next, Cp, Cn):
    return pl.pallas_call(
        _mid_tc,
        name=f"tc_mid{Cp}_{Cn}",
        grid=(GRID,),
        in_specs=[
            pl.BlockSpec((NC, BR, Cp), lambda i: (_c(0), i, _c(0))),
            pl.BlockSpec((BR, Cp), lambda i: (i, _c(0))),
            pl.BlockSpec((BR, 1), lambda i: (i, _c(0))),
            pl.BlockSpec((1, Cp), lambda i: (_c(0), _c(0))),
            pl.BlockSpec((Cp, Cn), lambda i: (_c(0), _c(0))),
        ],
        out_specs=pl.BlockSpec((BR, Cn), lambda i: (i, _c(0))),
        out_shape=jax.ShapeDtypeStruct((NPAD, Cn), jnp.float32),
    )(s_parts, g_prev, dinv, b_prev.reshape(1, Cp), W_next)


def _mid_split_tc(sa_ref, sb_ref, ga_ref, gb_ref, dinv_ref, b_ref, w_ref,
                  out_ref):
    sa = sa_ref[...]
    sb = sb_ref[...]
    s = jnp.concatenate([sa[0] + sa[1], sb[0] + sb[1]], axis=1)
    g = jnp.concatenate([ga_ref[...], gb_ref[...]], axis=1)
    dinv = dinv_ref[...]
    z = (s + g) * dinv + b_ref[...]
    z = jnp.maximum(z, 0.0)
    out_ref[...] = jnp.dot(z, w_ref[...],
                           preferred_element_type=jnp.float32) * dinv


def _mid_split(sa, sb, ga, gb, dinv, b_prev, W_next, Cp, Cn):
    ch = Cp // 2
    return pl.pallas_call(
        _mid_split_tc,
        name=f"tc_mids{Cp}_{Cn}",
        grid=(GRID,),
        in_specs=[
            pl.BlockSpec((NC, BR, ch), lambda i: (_c(0), i, _c(0))),
            pl.BlockSpec((NC, BR, ch), lambda i: (_c(0), i, _c(0))),
            pl.BlockSpec((BR, ch), lambda i: (i, _c(0))),
            pl.BlockSpec((BR, ch), lambda i: (i, _c(0))),
            pl.BlockSpec((BR, 1), lambda i: (i, _c(0))),
            pl.BlockSpec((1, Cp), lambda i: (_c(0), _c(0))),
            pl.BlockSpec((Cp, Cn), lambda i: (_c(0), _c(0))),
        ],
        out_specs=pl.BlockSpec((BR, Cn), lambda i: (i, _c(0))),
        out_shape=jax.ShapeDtypeStruct((NPAD, Cn), jnp.float32),
    )(sa, sb, ga, gb, dinv, b_prev.reshape(1, Cp), W_next)


def _final_tc(s_ref, g_ref, dinv_ref, b_ref, out_ref, acc_ref):
    i = pl.program_id(0)
    s = jnp.sum(s_ref[...], axis=0)[:, None]
    z = (s + g_ref[...]) * dinv_ref[...] + b_ref[...]
    z = jnp.maximum(z, 0.0)
    rows = lax.broadcasted_iota(jnp.int32, (BR, 1), 0) + i * _c(BR)
    z = jnp.where(rows < N, z, -jnp.inf)
    m = jnp.max(z)

    @pl.when(i == 0)
    def _():
        acc_ref[0, 0] = m

    @pl.when(i > 0)
    def _():
        acc_ref[0, 0] = jnp.maximum(acc_ref[0, 0], m)

    @pl.when(i == GRID - 1)
    def _():
        p = acc_ref[0, 0]
        v = p - (jnp.log(jnp.exp(p - p)) + p)
        out_ref[...] = jnp.broadcast_to(v, (1, 1))


def _final(s_parts, g4, dinv, b4):
    return pl.pallas_call(
        _final_tc,
        name="tc_final",
        grid=(GRID,),
        in_specs=[
            pl.BlockSpec((NW, BR), lambda i: (_c(0), i)),
            pl.BlockSpec((BR, 1), lambda i: (i, _c(0))),
            pl.BlockSpec((BR, 1), lambda i: (i, _c(0))),
            pl.BlockSpec((1, 1), lambda i: (_c(0), _c(0))),
        ],
        out_specs=pl.BlockSpec((1, 1), lambda i: (_c(0), _c(0))),
        out_shape=jax.ShapeDtypeStruct((1, 1), jnp.float32),
        scratch_shapes=[pltpu.SMEM((1, 1), jnp.float32)],
    )(s_parts, g4, dinv, b4.reshape(1, 1))


def kernel(x, edge_index, W1, b1, W2, b2, W3, b3, W4, b4):
    src = edge_index[0].astype(jnp.int32)
    dst = edge_index[1].astype(jnp.int32)
    pad = EPAD - E
    src3 = jnp.concatenate(
        [src, jnp.zeros((pad,), jnp.int32)]).reshape(NW, BLKS, EB)
    dst3 = jnp.concatenate(
        [dst, jnp.full((pad,), N, jnp.int32)]).reshape(NW, BLKS, EB)
    xp = jnp.concatenate(
        [x.astype(jnp.float32), jnp.zeros((NPAD - N, D), jnp.float32)], axis=0)

    deg_parts = _deg_call()(dst3)
    dinv, g1 = _stage1(deg_parts, xp, W1)
    s1 = _make_prop(16)(g1, src3, dst3)
    g2 = _mid(s1, g1, dinv, b1, W2, 16, 32)
    s2 = _make_prop(32)(g2, src3, dst3)
    g3a, g3b = _mid2(s2, g2, dinv, b2, W3, 32, 64)
    s3a, s3b = _prop64_call()(g3a, g3b, src3, dst3)
    g4 = _mid_split(s3a, s3b, g3a, g3b, dinv, b3, W4, 64, 1)
    s4 = _prop1_call()(g4.reshape(NPAD), src3, dst3)
    return _final(s4, g4, dinv, b4)
